# baseline (speedup 1.0000x reference)
import functools

import jax
import jax.numpy as jnp
from jax import lax
from jax.experimental import pallas as pl
from jax.experimental.pallas import tpu as pltpu
from jax.experimental.pallas import tpu_sc as plsc

_K = 16
_EPS = 1e-5
_RT = 256

_NC = 2
_NS = 16
_NW = _NC * _NS


def _scal(v):
    return jnp.squeeze(lax.slice(v, (0,), (1,)))


def _sc_body(xyzt, xtf, out, xv, yv, zv, sv, dbuf, sd, si, ibuf, gbuf, obuf,
             gsem, osem, *, B, N, C):
    rows_per = (B * N) // _NW
    tec_per_b = N // rows_per
    blocks = rows_per // 8
    nchunk = N // 16

    wid = lax.axis_index("s") * _NC + lax.axis_index("c")
    b = wid // tec_per_b
    q = wid % tec_per_b
    lbase = q * rows_per
    gbase = b * N + lbase

    pltpu.sync_copy(xyzt.at[b * 4 + 0], xv.at[pl.ds(0, N)])
    pltpu.sync_copy(xyzt.at[b * 4 + 1], yv.at[pl.ds(0, N)])
    pltpu.sync_copy(xyzt.at[b * 4 + 2], zv.at[pl.ds(0, N)])
    pltpu.sync_copy(xyzt.at[b * 4 + 3], sv.at[pl.ds(0, N)])

    def round_step(j, _):
        o = j * 16
        for ref_ in (xv, yv, zv):
            u = plsc.bitcast(ref_[pl.ds(o, 16)], jnp.int32)
            u = (u + jnp.int32(0x7FFF) + ((u >> 16) & 1)) & jnp.int32(-65536)
            ref_[pl.ds(o, 16)] = plsc.bitcast(u, jnp.float32)
        return 0

    lax.fori_loop(0, N // 16, round_step, 0)

    iota16 = lax.iota(jnp.int32, 16)
    inf = jnp.float32(jnp.inf)
    bigi = jnp.int32(2 * N)

    def select_block(s, par):
        row0 = s * 8 + lbase
        qx = [xv[pl.ds(row0 + rr, 16)][0] for rr in range(8)]
        qy = [yv[pl.ds(row0 + rr, 16)][0] for rr in range(8)]
        qz = [zv[pl.ds(row0 + rr, 16)][0] for rr in range(8)]
        qs = [sv[pl.ds(row0 + rr, 16)][0] for rr in range(8)]

        def dist_step(j, accs):
            o = j * 16
            xc = xv[pl.ds(o, 16)]
            yc = yv[pl.ds(o, 16)]
            zc = zv[pl.ds(o, 16)]
            sc_ = sv[pl.ds(o, 16)]
            nxt = []
            for rr in range(8):
                g = xc * qx[rr] + yc * qy[rr] + zc * qz[rr]
                d = (qs[rr] + sc_) - (g + g)
                dbuf[rr, pl.ds(o, 16)] = d
                nxt.append(jnp.minimum(accs[rr], d))
            return tuple(nxt)

        Rs = lax.fori_loop(0, N // 16, dist_step,
                           (jnp.full((16,), inf),) * 8)

        for rr in range(8):
            rs, _ = plsc.sort_key_val(Rs[rr], iota16)
            tau = rs[15]

            def compact_step(j, c):
                o = j * 16
                d = dbuf[rr, pl.ds(o, 16)]
                m = d <= tau
                plsc.store_compressed(sd.at[pl.ds(c, 16)], d, mask=m)
                plsc.store_compressed(si.at[pl.ds(c, 16)], iota16 + o, mask=m)
                return c + _scal(plsc.all_reduce_population_count(m))

            cnt = lax.fori_loop(0, N // 16, compact_step, jnp.int32(0))

            sd[pl.ds(cnt, 16)] = jnp.full((16,), inf)
            si[pl.ds(cnt, 16)] = jnp.full((16,), bigi)

            def merge_step(ms, kv):
                rk, ri = kv
                dv = sd[pl.ds(ms * 16, 16)]
                iv = si[pl.ds(ms * 16, 16)]
                dv, iv = plsc.sort_key_val(dv, iv)
                cr = lax.rev(dv, (0,))
                ir = lax.rev(iv, (0,))
                take = (rk < cr) | ((rk == cr) & (ri < ir))
                nk = jnp.where(take, rk, cr)
                ni = jnp.where(take, ri, ir)
                nk, ni = plsc.sort_key_val(nk, ni)
                return (nk, ni)

            nmerge = (cnt + 15) // 16
            _, ri = lax.fori_loop(0, nmerge, merge_step,
                                  (jnp.full((16,), inf),
                                   jnp.full((16,), bigi)))
            ibuf[par, pl.ds(rr * 16, 16)] = ri + b * N

    def gather_start(par):
        pltpu.async_copy(xtf.at[ibuf.at[par]], gbuf.at[par], gsem[par])

    def gather_wait(par):
        pltpu.make_async_copy(xtf.at[ibuf.at[par]], gbuf.at[par],
                              gsem[par]).wait()

    def sum_and_flush(s, par):
        def sum_row(rr, _2):
            for j in range(8):
                acc = gbuf[par, rr * 16, pl.ds(j * 16, 16)]
                for t in range(1, 16):
                    acc = acc + gbuf[par, rr * 16 + t, pl.ds(j * 16, 16)]
                obuf[par, rr, pl.ds(j * 16, 16)] = acc
            return 0

        lax.fori_loop(0, 8, sum_row, 0)
        pltpu.async_copy(obuf.at[par], out.at[pl.ds(gbase + s * 8, 8)],
                         osem[par])

    def flush_wait(par):
        pltpu.make_async_copy(obuf.at[par], out.at[pl.ds(gbase, 8)],
                              osem[par]).wait()

    select_block(0, 0)
    gather_start(0)

    def pair_step(p, _):
        select_block(2 * p + 1, 1)
        gather_start(1)

        @pl.when(p > 0)
        def _():
            flush_wait(0)

        gather_wait(0)
        sum_and_flush(2 * p, 0)

        @pl.when(p < blocks // 2 - 1)
        def _():
            select_block(2 * p + 2, 0)
            gather_start(0)

        @pl.when(p > 0)
        def _():
            flush_wait(1)

        gather_wait(1)
        sum_and_flush(2 * p + 1, 1)
        return 0

    lax.fori_loop(0, blocks // 2, pair_step, 0)
    flush_wait(0)
    flush_wait(1)


def sc_summed(xyz, x):
    B, C, N = x.shape
    xyztf = jnp.transpose(xyz, (0, 2, 1))
    sq = jnp.sum(xyztf * xyztf, axis=1, keepdims=True)
    xyzt = jnp.concatenate([xyztf, sq], axis=1).reshape(B * 4, N)
    xtf = jnp.transpose(x, (0, 2, 1)).reshape(B * N, C)

    mesh = plsc.VectorSubcoreMesh(core_axis_name="c", subcore_axis_name="s",
                                  num_cores=_NC, num_subcores=_NS)
    body = functools.partial(_sc_body, B=B, N=N, C=C)
    f = pl.kernel(
        body,
        out_type=jax.ShapeDtypeStruct((B * N, C), jnp.float32),
        mesh=mesh,
        compiler_params=pltpu.CompilerParams(
            needs_layout_passes=False,
            use_tc_tiling_on_sc=False,
        ),
        scratch_types=[
            pltpu.VMEM((N + 16,), jnp.float32),
            pltpu.VMEM((N + 16,), jnp.float32),
            pltpu.VMEM((N + 16,), jnp.float32),
            pltpu.VMEM((N + 16,), jnp.float32),
            pltpu.VMEM((8, N), jnp.float32),
            pltpu.VMEM((N + 16,), jnp.float32),
            pltpu.VMEM((N + 16,), jnp.int32),
            pltpu.VMEM((2, 128), jnp.int32),
            pltpu.VMEM((2, 128, C), jnp.float32),
            pltpu.VMEM((2, 8, C), jnp.float32),
            (pltpu.SemaphoreType.DMA, pltpu.SemaphoreType.DMA),
            (pltpu.SemaphoreType.DMA, pltpu.SemaphoreType.DMA),
        ],
    )
    return f(xyzt, xtf)


def _conv_block(sm_ref, x_ref, w_ref, bc_ref, h_ref, s1_ref, s2_ref):
    b = pl.program_id(0)
    t = pl.program_id(1)
    st = sm_ref[0]
    wst = jax.lax.dot_general(w_ref[...], st, (((1,), (1,)), ((), ())),
                              preferred_element_type=jnp.float32)
    wx = jax.lax.dot_general(w_ref[...], x_ref[0], (((1,), (0,)), ((), ())),
                             preferred_element_type=jnp.float32)
    h = jnp.maximum(wst - wx + bc_ref[...], 0.0)
    h_ref[0] = h
    s1 = jnp.sum(h, axis=1, keepdims=True)
    s2 = jnp.sum(h * h, axis=1, keepdims=True)

    @pl.when(jnp.logical_and(b == 0, t == 0))
    def _():
        s1_ref[...] = s1
        s2_ref[...] = s2

    @pl.when(jnp.logical_or(b != 0, t != 0))
    def _():
        s1_ref[...] = s1_ref[...] + s1
        s2_ref[...] = s2_ref[...] + s2


def _bn_block(x_ref, h_ref, s1_ref, s2_ref, g_ref, be_ref, o_ref, *, count):
    inv = jnp.float32(1.0 / count)
    mean = s1_ref[...] * inv
    var = s2_ref[...] * inv - mean * mean
    rstd = jax.lax.rsqrt(var + _EPS)
    scale = g_ref[...] * rstd
    shift = be_ref[...] - mean * scale
    o_ref[0] = x_ref[0] + h_ref[0] * scale + shift


def kernel(xyz, x, conv_w, conv_b, gamma, beta):
    B, C, N = x.shape
    NT = N // _RT
    summed = sc_summed(xyz, x).reshape(B, N, C)
    bc = conv_b.reshape(C, 1)
    gc = gamma.reshape(C, 1)
    bec = beta.reshape(C, 1)

    h, s1, s2 = pl.pallas_call(
        _conv_block,
        grid=(B, NT),
        in_specs=[
            pl.BlockSpec((1, _RT, C), lambda b, t: (b, t, 0)),
            pl.BlockSpec((1, C, _RT), lambda b, t: (b, 0, t)),
            pl.BlockSpec((C, C), lambda b, t: (0, 0)),
            pl.BlockSpec((C, 1), lambda b, t: (0, 0)),
        ],
        out_specs=[
            pl.BlockSpec((1, C, _RT), lambda b, t: (b, 0, t)),
            pl.BlockSpec((C, 1), lambda b, t: (0, 0)),
            pl.BlockSpec((C, 1), lambda b, t: (0, 0)),
        ],
        out_shape=[
            jax.ShapeDtypeStruct((B, C, N), jnp.float32),
            jax.ShapeDtypeStruct((C, 1), jnp.float32),
            jax.ShapeDtypeStruct((C, 1), jnp.float32),
        ],
    )(summed, x, conv_w, bc)

    out = pl.pallas_call(
        functools.partial(_bn_block, count=B * N),
        grid=(B,),
        in_specs=[
            pl.BlockSpec((1, C, N), lambda b: (b, 0, 0)),
            pl.BlockSpec((1, C, N), lambda b: (b, 0, 0)),
            pl.BlockSpec((C, 1), lambda b: (0, 0)),
            pl.BlockSpec((C, 1), lambda b: (0, 0)),
            pl.BlockSpec((C, 1), lambda b: (0, 0)),
            pl.BlockSpec((C, 1), lambda b: (0, 0)),
        ],
        out_specs=pl.BlockSpec((1, C, N), lambda b: (b, 0, 0)),
        out_shape=jax.ShapeDtypeStruct((B, C, N), jnp.float32),
    )(x, h, s1, s2, gc, bec)
    return out
